# Initial kernel scaffold; baseline (speedup 1.0000x reference)
import jax
import jax.numpy as jnp
from jax.experimental import pallas as pl


def kernel(x, edge_index, W_rel1, W_root1, b1, W_rel2, W_root2, b2, W_rel3, W_root3, b3, g1, be1, g2, be2, W_lin, b_lin):
    raise NotImplementedError("write your pallas kernel here")


# speedup vs baseline: 7.2616x; 7.2616x over previous
import jax
import jax.numpy as jnp
from jax import lax
from jax.experimental import pallas as pl
from jax.experimental.pallas import tpu as pltpu
from jax.experimental.pallas import tpu_sc as plsc

_N = 10000
_E = 320000
_D = 128
_NC = 2
_NS = 16
_NW = _NC * _NS
_EPW = _E // _NW
_CH = 80
_NCHUNK = _EPW // _CH
_RPT = 632
_NPAD = _RPT * _NS


def _segsum_body(h_hbm, srcg_hbm, dstg_hbm, zeros_hbm, out_hbm,
                 src_v, dst_v, rows_v, acc_sh, sem):
    cid = lax.axis_index("c")
    sid = lax.axis_index("s")
    wid = cid * _NS + sid
    pltpu.sync_copy(zeros_hbm, acc_sh.at[pl.ds(sid * _RPT, _RPT)])
    pltpu.sync_copy(srcg_hbm.at[wid], src_v)
    pltpu.sync_copy(dstg_hbm.at[wid], dst_v)
    plsc.subcore_barrier()

    def step(j, carry):
        pltpu.async_copy(h_hbm.at[src_v.at[j]], rows_v, sem).wait()
        pltpu.sync_copy(rows_v, acc_sh.at[dst_v.at[j]], add=True)
        return carry

    lax.fori_loop(0, _NCHUNK, step, 0)
    plsc.subcore_barrier()
    pltpu.sync_copy(acc_sh.at[pl.ds(sid * _RPT, _RPT)],
                    out_hbm.at[cid].at[pl.ds(sid * _RPT, _RPT)])


_segsum = pl.kernel(
    _segsum_body,
    mesh=plsc.VectorSubcoreMesh(core_axis_name="c", subcore_axis_name="s"),
    out_type=jax.ShapeDtypeStruct((_NC, _NPAD, _D), jnp.float32),
    scratch_types=[
        pltpu.VMEM((_NCHUNK, _CH), jnp.int32),
        pltpu.VMEM((_NCHUNK, _CH), jnp.int32),
        pltpu.VMEM((_CH, _D), jnp.float32),
        pltpu.VMEM_SHARED((_NPAD, _D), jnp.float32),
        pltpu.SemaphoreType.DMA,
    ],
)


def _layer_body(p_ref, h_ref, wr_ref, wo_ref, b_ref, g_ref, be_ref, o_ref):
    aggr = p_ref[0, :_N] + p_ref[1, :_N]
    t = (jnp.dot(aggr, wr_ref[...], preferred_element_type=jnp.float32)
         + jnp.dot(h_ref[...], wo_ref[...], preferred_element_type=jnp.float32)
         + b_ref[...])
    mu = jnp.mean(t, axis=0, keepdims=True)
    var = jnp.mean((t - mu) ** 2, axis=0, keepdims=True)
    y = (t - mu) * lax.rsqrt(var + 1e-5) * g_ref[...] + be_ref[...]
    o_ref[...] = jnp.where(y >= 0.0, y, 0.01 * y)


_layer_call = pl.pallas_call(
    _layer_body,
    out_shape=jax.ShapeDtypeStruct((_N, _D), jnp.float32),
)


def _final_body(p_ref, h_ref, wr_ref, wo_ref, b_ref, wl_ref, bl_ref, o_ref):
    aggr = p_ref[0, :_N] + p_ref[1, :_N]
    t = (jnp.dot(aggr, wr_ref[...], preferred_element_type=jnp.float32)
         + jnp.dot(h_ref[...], wo_ref[...], preferred_element_type=jnp.float32)
         + b_ref[...])
    o_ref[...] = (jnp.dot(t, wl_ref[...], preferred_element_type=jnp.float32)
                  + bl_ref[...])


_final_call = pl.pallas_call(
    _final_body,
    out_shape=jax.ShapeDtypeStruct((_N, _D), jnp.float32),
)


def kernel(x, edge_index, W_rel1, W_root1, b1, W_rel2, W_root2, b2,
           W_rel3, W_root3, b3, g1, be1, g2, be2, W_lin, b_lin):
    srcg = edge_index[0].reshape(_NW, _NCHUNK, _CH)
    dstg = edge_index[1].reshape(_NW, _NCHUNK, _CH)
    zeros = jnp.zeros((_RPT, _D), jnp.float32)

    p1 = _segsum(x, srcg, dstg, zeros)
    h1 = _layer_call(p1, x, W_rel1, W_root1, b1.reshape(1, _D),
                     g1.reshape(1, _D), be1.reshape(1, _D))
    p2 = _segsum(h1, srcg, dstg, zeros)
    h2 = _layer_call(p2, h1, W_rel2, W_root2, b2.reshape(1, _D),
                     g2.reshape(1, _D), be2.reshape(1, _D))
    p3 = _segsum(h2, srcg, dstg, zeros)
    return _final_call(p3, h2, W_rel3, W_root3, b3.reshape(1, _D),
                       W_lin, b_lin.reshape(1, _D))
